# baseline (speedup 1.0000x reference)
import functools
import jax
import jax.numpy as jnp
from jax import lax
from jax.experimental import pallas as pl
from jax.experimental.pallas import tpu as pltpu
from jax.experimental.pallas import tpu_sc as plsc

N = 10000
D = 128
G = 16
E = 320000

NC = 2
NS = 16
NW = NC * NS
K = 128
CH = 80
CH2 = CH // 2
EPW = K * CH
EP = EPW * NW
NP = 10240
RPT = NP // NS

_mesh = plsc.VectorSubcoreMesh(core_axis_name="c", subcore_axis_name="s")


@functools.partial(
    pl.kernel,
    out_type=jax.ShapeDtypeStruct((NC * NP, D), jnp.float32),
    mesh=_mesh,
    scratch_types=[
        pltpu.VMEM((CH, K), jnp.int32),
        pltpu.VMEM((K, D), jnp.float32),
        pltpu.VMEM_SHARED((NP, D), jnp.float32),
        pltpu.SemaphoreType.DMA,
        pltpu.SemaphoreType.DMA,
    ],
)
def _deg_kernel(dst_hbm, out_hbm, dst_v, ones_v, acc_sh, sem0, sem1):
    c = lax.axis_index("c")
    s = lax.axis_index("s")
    wid = s * NC + c

    idx_cp = pltpu.async_copy(dst_hbm.at[pl.ds(wid * CH, CH)], dst_v, sem0)

    zero = jnp.zeros((16,), jnp.float32)

    def fill_zero_row(i, _):
        def fill_lane(j, _):
            ones_v[i, pl.ds(j * 16, 16)] = zero
            return 0
        lax.fori_loop(0, D // 16, fill_lane, 0)
        return 0
    lax.fori_loop(0, K, fill_zero_row, 0)

    def zero_acc(i, _):
        pltpu.sync_copy(ones_v, acc_sh.at[pl.ds(s * RPT + i * K, K)])
        return 0
    lax.fori_loop(0, RPT // K, zero_acc, 0)

    one = jnp.full((16,), 1.0, jnp.float32)

    def fill_one_row(i, _):
        def fill_lane(j, _):
            ones_v[i, pl.ds(j * 16, 16)] = one
            return 0
        lax.fori_loop(0, D // 16, fill_lane, 0)
        return 0
    lax.fori_loop(0, K, fill_one_row, 0)
    idx_cp.wait()
    plsc.subcore_barrier()

    pltpu.async_copy(ones_v, acc_sh.at[dst_v.at[0]], sem0, add=True)

    def body(j, _):
        i = 2 * j
        pltpu.async_copy(ones_v, acc_sh.at[dst_v.at[i + 1]], sem1, add=True)
        pltpu.make_async_copy(ones_v, acc_sh.at[dst_v.at[i]], sem0).wait()

        @pl.when(j < CH2 - 1)
        def _():
            pltpu.async_copy(ones_v, acc_sh.at[dst_v.at[i + 2]], sem0, add=True)
        pltpu.make_async_copy(ones_v, acc_sh.at[dst_v.at[i + 1]], sem1).wait()
        return 0
    lax.fori_loop(0, CH2, body, 0)

    plsc.subcore_barrier()

    def copy_out(i, _):
        pltpu.sync_copy(acc_sh.at[pl.ds(s * RPT + i * K, K)], ones_v)
        pltpu.sync_copy(ones_v, out_hbm.at[pl.ds(c * NP + s * RPT + i * K, K)])
        return 0
    lax.fori_loop(0, RPT // K, copy_out, 0)


@functools.partial(
    pl.kernel,
    out_type=jax.ShapeDtypeStruct((NC * NP, D), jnp.float32),
    mesh=_mesh,
    scratch_types=[
        pltpu.VMEM((K,), jnp.int32),
        pltpu.VMEM((K,), jnp.int32),
        pltpu.VMEM((CH, K), jnp.int32),
        pltpu.VMEM((K, D), jnp.float32),
        pltpu.VMEM((K, D), jnp.float32),
        pltpu.VMEM_SHARED((NP, D), jnp.float32),
        pltpu.SemaphoreType.DMA,
        pltpu.SemaphoreType.DMA,
        pltpu.SemaphoreType.DMA,
        pltpu.SemaphoreType.DMA,
    ],
)
def _agg_kernel(hp_hbm, src_hbm, dst_hbm, out_hbm,
                sidx0, sidx1, dst_v, buf0, buf1, acc_sh, si0, si1, g0, g1):
    c = lax.axis_index("c")
    s = lax.axis_index("s")
    wid = s * NC + c
    ebase = wid * EPW

    icp = pltpu.async_copy(dst_hbm.at[pl.ds(wid * CH, CH)], dst_v, g1)

    zero = jnp.zeros((16,), jnp.float32)

    def fill_zero_row(i, _):
        def fill_lane(j, _):
            buf0[i, pl.ds(j * 16, 16)] = zero
            return 0
        lax.fori_loop(0, D // 16, fill_lane, 0)
        return 0
    lax.fori_loop(0, K, fill_zero_row, 0)

    def zero_acc(i, _):
        pltpu.sync_copy(buf0, acc_sh.at[pl.ds(s * RPT + i * K, K)])
        return 0
    lax.fori_loop(0, RPT // K, zero_acc, 0)
    icp.wait()
    plsc.subcore_barrier()

    pltpu.async_copy(src_hbm.at[pl.ds(ebase, K)], sidx0, si0)
    pltpu.async_copy(src_hbm.at[pl.ds(ebase + K, K)], sidx1, si1)
    pltpu.make_async_copy(src_hbm.at[pl.ds(ebase, K)], sidx0, si0).wait()
    pltpu.async_copy(hp_hbm.at[sidx0], buf0, g0)

    def body(j, _):
        i = 2 * j
        pltpu.make_async_copy(src_hbm.at[pl.ds(ebase, K)], sidx1, si1).wait()
        pltpu.async_copy(hp_hbm.at[sidx1], buf1, g1)
        pltpu.make_async_copy(hp_hbm.at[sidx0], buf0, g0).wait()

        @pl.when(j < CH2 - 1)
        def _():
            pltpu.async_copy(src_hbm.at[pl.ds(ebase + (i + 2) * K, K)],
                             sidx0, si0)
        pltpu.sync_copy(buf0, acc_sh.at[dst_v.at[i]], add=True)
        pltpu.make_async_copy(hp_hbm.at[sidx1], buf1, g1).wait()

        @pl.when(j < CH2 - 1)
        def _():
            pltpu.make_async_copy(src_hbm.at[pl.ds(ebase, K)], sidx0, si0).wait()
            pltpu.async_copy(hp_hbm.at[sidx0], buf0, g0)
            pltpu.async_copy(src_hbm.at[pl.ds(ebase + (i + 3) * K, K)],
                             sidx1, si1)
        pltpu.sync_copy(buf1, acc_sh.at[dst_v.at[i + 1]], add=True)
        return 0
    lax.fori_loop(0, CH2, body, 0)

    plsc.subcore_barrier()

    def copy_out(i, _):
        pltpu.sync_copy(acc_sh.at[pl.ds(s * RPT + i * K, K)], buf0)
        pltpu.sync_copy(buf0, out_hbm.at[pl.ds(c * NP + s * RPT + i * K, K)])
        return 0
    lax.fori_loop(0, RPT // K, copy_out, 0)


def _mm_scale_body(deg0_ref, deg1_ref, x_ref, w_ref, hp_ref, dinv_ref):
    deg = deg0_ref[...][:, 0:1] + deg1_ref[...][:, 0:1] + 1.0
    dinv = lax.rsqrt(deg)
    h = jnp.dot(x_ref[...], w_ref[...], preferred_element_type=jnp.float32)
    hp_ref[...] = h * dinv
    dinv_ref[...] = jnp.broadcast_to(dinv, (N, D))


_mm_scale = pl.pallas_call(
    _mm_scale_body,
    out_shape=(jax.ShapeDtypeStruct((N, D), jnp.float32),
               jax.ShapeDtypeStruct((N, D), jnp.float32)),
)


def _combine_mm_body(s0_ref, s1_ref, hp_ref, dinv_ref, w_ref, b_ref, out_ref):
    agg = s0_ref[...] + s1_ref[...] + hp_ref[...]
    o1 = dinv_ref[...] * agg + b_ref[...]
    out_ref[...] = dinv_ref[...] * jnp.dot(
        o1, w_ref[...], preferred_element_type=jnp.float32)


_combine_mm = pl.pallas_call(
    _combine_mm_body,
    out_shape=jax.ShapeDtypeStruct((N, D), jnp.float32),
)


def _final_body(s0_ref, s1_ref, hp_ref, dinv_ref, b_ref, batch_ref, out_ref):
    agg = s0_ref[...] + s1_ref[...] + hp_ref[...]
    o2 = dinv_ref[...] * agg + b_ref[...]
    ids = batch_ref[...]
    gid = lax.broadcasted_iota(jnp.int32, (G, N), 0)
    p = (gid == ids).astype(jnp.float32)
    ssum = jnp.dot(p, o2, preferred_element_type=jnp.float32)
    cnt = jnp.sum(p, axis=1, keepdims=True)
    out_ref[...] = ssum / jnp.maximum(cnt, 1.0)


_final = pl.pallas_call(
    _final_body,
    out_shape=jax.ShapeDtypeStruct((G, D), jnp.float32),
)


def kernel(x, edge_index, batch, W1, b1, W2, b2):
    pad = EP - E
    src = jnp.concatenate([edge_index[0], jnp.zeros((pad,), jnp.int32)])
    dst = jnp.concatenate([edge_index[1], jnp.full((pad,), N, jnp.int32)])
    dst3 = dst.reshape(NW * CH, K)

    degp = _deg_kernel(dst3)
    deg0 = degp[:N]
    deg1 = degp[NP:NP + N]

    hp1, dinv = _mm_scale(deg0, deg1, x, W1)
    s1 = _agg_kernel(hp1, src, dst3)
    hp2 = _combine_mm(s1[:N], s1[NP:NP + N], hp1, dinv, W2,
                      b1.reshape(1, D))
    s2 = _agg_kernel(hp2, src, dst3)
    return _final(s2[:N], s2[NP:NP + N], hp2, dinv, b2.reshape(1, D),
                  batch.reshape(1, N))
